# Initial kernel scaffold; baseline (speedup 1.0000x reference)
import jax
import jax.numpy as jnp
from jax.experimental import pallas as pl


def kernel(logits, targets):
    raise NotImplementedError("write your pallas kernel here")


# speedup vs baseline: 6.5985x; 6.5985x over previous
import jax
import jax.numpy as jnp
from jax.experimental import pallas as pl
from jax.experimental.pallas import tpu as pltpu

_N_CATS = 10
_CE_W = 1.0
_QWK_W = 0.5
_ORD_W = 0.3


def _loss_body(x_ref, t_ref, out_ref):
    x = x_ref[...]
    t = t_ref[...]
    n_tok = x.shape[1]
    tf = t.astype(jnp.float32)

    m = jnp.max(x, axis=0, keepdims=True)
    e = jnp.exp(x - m)
    s = jnp.sum(e, axis=0, keepdims=True)
    lse = m + jnp.log(s)

    cats = jax.lax.broadcasted_iota(jnp.int32, x.shape, 0).astype(jnp.float32)
    x_at_t = jnp.sum(jnp.where(cats == tf, x, 0.0), axis=0, keepdims=True)
    ce = lse - x_at_t

    pred = jnp.sum(cats * e, axis=0, keepdims=True) / s
    pen = jnp.abs(pred - tf)
    acc1 = jnp.sum(ce + _ORD_W * pen)

    p = jnp.min(jnp.where(x == m, cats, jnp.float32(_N_CATS)), axis=0,
                keepdims=True)

    maskf = (t > 0).astype(jnp.float32)
    n = jnp.sum(maskf)
    d = tf - p
    sumsq = jnp.sum(maskf * d * d)
    s1t = jnp.sum(maskf * tf)
    s2t = jnp.sum(maskf * tf * tf)
    s1p = jnp.sum(maskf * p)
    s2p = jnp.sum(maskf * p * p)

    ce_loss = acc1 / n_tok
    nm = jnp.maximum(n, 1.0)
    inv_w = 1.0 / ((_N_CATS - 1.0) ** 2)
    numer = (n - sumsq * inv_w) / nm
    denom = (n * n - (n * s2t + n * s2p - 2.0 * s1t * s1p) * inv_w) / (nm * nm)
    qwk = jnp.where(denom == 0.0, 0.0, numer / jnp.where(denom == 0.0, 1.0, denom))
    qwk = jnp.where(n == 0.0, 0.0, qwk)
    qwk_loss = jnp.where(n == 0.0, 0.0, 1.0 - qwk)
    total = _CE_W * ce_loss + _QWK_W * qwk_loss

    out_ref[0] = total
    out_ref[1] = ce_loss
    out_ref[2] = qwk_loss


def kernel(logits, targets):
    b, s, c = logits.shape
    n_tok = b * s
    xt = logits.reshape(n_tok, c).T
    tr = targets.reshape(1, n_tok)

    out = pl.pallas_call(
        _loss_body,
        out_shape=jax.ShapeDtypeStruct((3,), jnp.float32),
        out_specs=pl.BlockSpec(memory_space=pltpu.SMEM),
    )(xt, tr)
    return (out[0], out[1], out[2])
